# baseline (speedup 1.0000x reference)
import functools

import jax
import jax.numpy as jnp
from jax import lax
from jax.experimental import pallas as pl
from jax.experimental.pallas import tpu as pltpu
from jax.experimental.pallas import tpu_sc as plsc

VOCAB = 1000000
EMBED_DIM = 64
UNITS = 128
BATCH = 1024
SEQ = 50

_NC = 2
_NS = 16
_NW = _NC * _NS
_ROWS = BATCH * SEQ
_RPW = _ROWS // _NW
_CW = 80
_CH = _RPW // _CW


@functools.lru_cache(maxsize=1)
def _make_sc_gather():
    mesh = plsc.VectorSubcoreMesh(core_axis_name="c", subcore_axis_name="s")

    @functools.partial(
        pl.kernel,
        mesh=mesh,
        out_type=jax.ShapeDtypeStruct((_NW, _RPW, EMBED_DIM), jnp.float32),
        scratch_types=[
            pltpu.VMEM((_CH, _CW), jnp.int32),
            pltpu.VMEM((_RPW, EMBED_DIM), jnp.float32),
            pltpu.SemaphoreType.DMA,
        ],
        compiler_params=pltpu.CompilerParams(use_tc_tiling_on_sc=False),
    )
    def sc_gather(table_hbm, idx_hbm, out_hbm, idx_v, rows_v, sem):
        wid = lax.axis_index("s") * _NC + lax.axis_index("c")
        pltpu.sync_copy(idx_hbm.at[wid], idx_v)
        copies = []
        for j in range(_CH):
            copies.append(
                pltpu.async_copy(
                    table_hbm.at[idx_v.at[j]],
                    rows_v.at[pl.ds(j * _CW, _CW)],
                    sem,
                )
            )
        for cp in copies:
            cp.wait()
        pltpu.sync_copy(rows_v, out_hbm.at[wid])

    return sc_gather


_TB = 8
_NTB = (SEQ + _TB - 1) // _TB


def _gru_body(emb_ref, h0_ref, wk_ref, wr_ref, bi_ref, br_ref, out_ref, h_ref):
    tb = pl.program_id(0)

    @pl.when(tb == 0)
    def _():
        h_ref[...] = h0_ref[...]

    h = h_ref[...]
    wk = wk_ref[...]
    wr = wr_ref[...]
    bi = bi_ref[...]
    br = br_ref[...]
    for t in range(_TB):
        xt = emb_ref[t]
        matx = jnp.dot(xt, wk, preferred_element_type=jnp.float32) + bi
        math = jnp.dot(h, wr, preferred_element_type=jnp.float32) + br
        xz = matx[:, 0:UNITS]
        xr = matx[:, UNITS:2 * UNITS]
        xh = matx[:, 2 * UNITS:3 * UNITS]
        hz = math[:, 0:UNITS]
        hr = math[:, UNITS:2 * UNITS]
        hh_rec = math[:, 2 * UNITS:3 * UNITS]
        z = jax.nn.sigmoid(xz + hz)
        r = jax.nn.sigmoid(xr + hr)
        hh = jnp.tanh(xh + r * hh_rec)
        h = z * h + (1.0 - z) * hh
        out_ref[:, t, :] = h
    h_ref[...] = h


def _gru_scan(emb, h0, wk, wr, bi, br):
    return pl.pallas_call(
        _gru_body,
        grid=(_NTB,),
        in_specs=[
            pl.BlockSpec((_TB, BATCH, EMBED_DIM), lambda t: (t, 0, 0)),
            pl.BlockSpec((BATCH, UNITS), lambda t: (0, 0)),
            pl.BlockSpec((EMBED_DIM, 3 * UNITS), lambda t: (0, 0)),
            pl.BlockSpec((UNITS, 3 * UNITS), lambda t: (0, 0)),
            pl.BlockSpec((1, 3 * UNITS), lambda t: (0, 0)),
            pl.BlockSpec((1, 3 * UNITS), lambda t: (0, 0)),
        ],
        out_specs=pl.BlockSpec((BATCH, _TB, UNITS), lambda t: (0, t, 0)),
        out_shape=jax.ShapeDtypeStruct((BATCH, SEQ, UNITS), jnp.float32),
        scratch_shapes=[pltpu.VMEM((BATCH, UNITS), jnp.float32)],
        compiler_params=pltpu.CompilerParams(
            vmem_limit_bytes=100 * 1024 * 1024,
        ),
    )(emb, h0, wk, wr, bi, br)


def kernel(x, gru_init_state, embedding, kernel, recurrent_kernel, bias_input, bias_recurrent):
    idx = jnp.transpose(x.astype(jnp.int32), (1, 0)).reshape(_NW, _CH, _CW)
    rows = _make_sc_gather()(embedding, idx)
    emb = rows.reshape(SEQ, BATCH, EMBED_DIM)

    output = _gru_scan(
        emb,
        gru_init_state,
        kernel,
        recurrent_kernel,
        bias_input.reshape(1, 3 * UNITS),
        bias_recurrent.reshape(1, 3 * UNITS),
    )
    state = output[:, SEQ - 1, :]
    return (output, state)


# speedup vs baseline: 1.7093x; 1.7093x over previous
import functools

import jax
import jax.numpy as jnp
from jax import lax
from jax.experimental import pallas as pl
from jax.experimental.pallas import tpu as pltpu
from jax.experimental.pallas import tpu_sc as plsc

VOCAB = 1000000
EMBED_DIM = 64
UNITS = 128
BATCH = 1024
SEQ = 50

_NC = 2
_NS = 16
_NW = _NC * _NS
_ROWS = BATCH * SEQ
_RPW = _ROWS // _NW
_FLUSH = 400
_NFL = _RPW // _FLUSH
_G = 16
_RINGG = 3


@functools.lru_cache(maxsize=1)
def _make_sc_gather():
    mesh = plsc.VectorSubcoreMesh(core_axis_name="c", subcore_axis_name="s")

    @functools.partial(
        pl.kernel,
        mesh=mesh,
        out_type=jax.ShapeDtypeStruct((_ROWS, EMBED_DIM), jnp.float32),
        scratch_types=[
            pltpu.VMEM((_RPW,), jnp.int32),
            pltpu.VMEM((_FLUSH, EMBED_DIM), jnp.float32),
            pltpu.SemaphoreType.DMA,
        ],
        compiler_params=pltpu.CompilerParams(use_tc_tiling_on_sc=True),
    )
    def sc_gather(table_hbm, idx_hbm, out_hbm, idx_v, rows_v, sem):
        wid = lax.axis_index("s") * _NC + lax.axis_index("c")
        base = wid * _RPW
        pltpu.sync_copy(idx_hbm.at[pl.ds(base, _RPW)], idx_v)

        def drain_group():
            pltpu.make_async_copy(
                table_hbm.at[pl.ds(0, _G)], rows_v.at[pl.ds(0, _G)], sem
            ).wait()

        for c in range(_NFL):
            cbase = c * _FLUSH

            def fire(g, carry, cbase=cbase):
                vec = idx_v[pl.ds(cbase + g * _G, _G)]
                for j in range(_G):
                    i = vec[j]
                    pltpu.async_copy(
                        table_hbm.at[pl.ds(i, 1)],
                        rows_v.at[pl.ds(g * _G + j, 1)],
                        sem,
                    )

                @pl.when(g >= _RINGG)
                def _():
                    drain_group()

                return carry

            lax.fori_loop(0, _FLUSH // _G, fire, 0)

            for _ in range(_RINGG):
                drain_group()
            pltpu.sync_copy(rows_v, out_hbm.at[pl.ds(base + c * _FLUSH, _FLUSH)])

    return sc_gather


_TB = 8
_NTB = (SEQ + _TB - 1) // _TB


def _gru_body(emb_ref, h0_ref, wk_ref, wr_ref, bi_ref, br_ref, out_ref, h_ref):
    tb = pl.program_id(0)

    @pl.when(tb == 0)
    def _():
        h_ref[...] = h0_ref[...]

    h = h_ref[...]
    wk = wk_ref[...]
    wr = wr_ref[...]
    bi = bi_ref[...]
    br = br_ref[...]
    for t in range(_TB):
        xt = emb_ref[t]
        matx = jnp.dot(xt, wk, preferred_element_type=jnp.float32) + bi
        math = jnp.dot(h, wr, preferred_element_type=jnp.float32) + br
        xz = matx[:, 0:UNITS]
        xr = matx[:, UNITS:2 * UNITS]
        xh = matx[:, 2 * UNITS:3 * UNITS]
        hz = math[:, 0:UNITS]
        hr = math[:, UNITS:2 * UNITS]
        hh_rec = math[:, 2 * UNITS:3 * UNITS]
        z = jax.nn.sigmoid(xz + hz)
        r = jax.nn.sigmoid(xr + hr)
        hh = jnp.tanh(xh + r * hh_rec)
        h = z * h + (1.0 - z) * hh
        out_ref[t] = h
    h_ref[...] = h


def _gru_scan(emb, h0, wk, wr, bi, br):
    return pl.pallas_call(
        _gru_body,
        grid=(_NTB,),
        in_specs=[
            pl.BlockSpec((_TB, BATCH, EMBED_DIM), lambda t: (t, 0, 0)),
            pl.BlockSpec((BATCH, UNITS), lambda t: (0, 0)),
            pl.BlockSpec((EMBED_DIM, 3 * UNITS), lambda t: (0, 0)),
            pl.BlockSpec((UNITS, 3 * UNITS), lambda t: (0, 0)),
            pl.BlockSpec((1, 3 * UNITS), lambda t: (0, 0)),
            pl.BlockSpec((1, 3 * UNITS), lambda t: (0, 0)),
        ],
        out_specs=pl.BlockSpec((_TB, BATCH, UNITS), lambda t: (t, 0, 0)),
        out_shape=jax.ShapeDtypeStruct((SEQ, BATCH, UNITS), jnp.float32),
        scratch_shapes=[pltpu.VMEM((BATCH, UNITS), jnp.float32)],
        compiler_params=pltpu.CompilerParams(
            vmem_limit_bytes=100 * 1024 * 1024,
        ),
    )(emb, h0, wk, wr, bi, br)


def kernel(x, gru_init_state, embedding, kernel, recurrent_kernel, bias_input, bias_recurrent):
    idx = jnp.transpose(x.astype(jnp.int32), (1, 0)).reshape(_ROWS)
    rows = _make_sc_gather()(embedding, idx)
    emb = rows.reshape(SEQ, BATCH, EMBED_DIM)

    out_tbu = _gru_scan(
        emb,
        gru_init_state,
        kernel,
        recurrent_kernel,
        bias_input.reshape(1, 3 * UNITS),
        bias_recurrent.reshape(1, 3 * UNITS),
    )
    output = jnp.transpose(out_tbu, (1, 0, 2))
    state = out_tbu[SEQ - 1]
    return (output, state)


# speedup vs baseline: 1.7418x; 1.0191x over previous
import functools

import jax
import jax.numpy as jnp
from jax import lax
from jax.experimental import pallas as pl
from jax.experimental.pallas import tpu as pltpu
from jax.experimental.pallas import tpu_sc as plsc

VOCAB = 1000000
EMBED_DIM = 64
UNITS = 128
BATCH = 1024
SEQ = 50

_NC = 2
_NS = 16
_NW = _NC * _NS
_ROWS = BATCH * SEQ
_RPW = _ROWS // _NW
_FLUSH = 800
_NFL = _RPW // _FLUSH
_G = 16
_RINGG = 6


@functools.lru_cache(maxsize=1)
def _make_sc_gather():
    mesh = plsc.VectorSubcoreMesh(core_axis_name="c", subcore_axis_name="s")

    @functools.partial(
        pl.kernel,
        mesh=mesh,
        out_type=jax.ShapeDtypeStruct((_ROWS, EMBED_DIM), jnp.float32),
        scratch_types=[
            pltpu.VMEM((_RPW,), jnp.int32),
            pltpu.VMEM((_FLUSH, EMBED_DIM), jnp.float32),
            pltpu.SemaphoreType.DMA,
        ],
        compiler_params=pltpu.CompilerParams(use_tc_tiling_on_sc=True),
    )
    def sc_gather(table_hbm, idx_hbm, out_hbm, idx_v, rows_v, sem):
        wid = lax.axis_index("s") * _NC + lax.axis_index("c")
        base = wid * _RPW
        pltpu.sync_copy(idx_hbm.at[pl.ds(base, _RPW)], idx_v)

        def drain_group():
            pltpu.make_async_copy(
                table_hbm.at[pl.ds(0, _G)], rows_v.at[pl.ds(0, _G)], sem
            ).wait()

        for c in range(_NFL):
            cbase = c * _FLUSH

            def fire(g, carry, cbase=cbase):
                vec = idx_v[pl.ds(cbase + g * _G, _G)]
                for j in range(_G):
                    i = vec[j]
                    pltpu.async_copy(
                        table_hbm.at[pl.ds(i, 1)],
                        rows_v.at[pl.ds(g * _G + j, 1)],
                        sem,
                    )

                @pl.when(g >= _RINGG)
                def _():
                    drain_group()

                return carry

            lax.fori_loop(0, _FLUSH // _G, fire, 0)

            for _ in range(_RINGG):
                drain_group()
            pltpu.sync_copy(rows_v, out_hbm.at[pl.ds(base + c * _FLUSH, _FLUSH)])

    return sc_gather


_TB = 8
_NTB = (SEQ + _TB - 1) // _TB


def _gru_body(emb_ref, h0_ref, wk_ref, wr_ref, bi_ref, br_ref, out_ref, h_ref):
    tb = pl.program_id(0)

    @pl.when(tb == 0)
    def _():
        h_ref[...] = h0_ref[...]

    h = h_ref[...]
    wk = wk_ref[...]
    wr = wr_ref[...]
    bi = bi_ref[...]
    br = br_ref[...]
    for t in range(_TB):
        xt = emb_ref[t]
        matx = jnp.dot(
            xt.astype(jnp.bfloat16), wk, preferred_element_type=jnp.float32
        ) + bi
        math = jnp.dot(
            h.astype(jnp.bfloat16), wr, preferred_element_type=jnp.float32
        ) + br
        xz = matx[:, 0:UNITS]
        xr = matx[:, UNITS:2 * UNITS]
        xh = matx[:, 2 * UNITS:3 * UNITS]
        hz = math[:, 0:UNITS]
        hr = math[:, UNITS:2 * UNITS]
        hh_rec = math[:, 2 * UNITS:3 * UNITS]
        z = jax.nn.sigmoid(xz + hz)
        r = jax.nn.sigmoid(xr + hr)
        hh = jnp.tanh(xh + r * hh_rec)
        h = z * h + (1.0 - z) * hh
        out_ref[t] = h
    h_ref[...] = h


def _gru_scan(emb, h0, wk, wr, bi, br):
    return pl.pallas_call(
        _gru_body,
        grid=(_NTB,),
        in_specs=[
            pl.BlockSpec((_TB, BATCH, EMBED_DIM), lambda t: (t, 0, 0)),
            pl.BlockSpec((BATCH, UNITS), lambda t: (0, 0)),
            pl.BlockSpec((EMBED_DIM, 3 * UNITS), lambda t: (0, 0)),
            pl.BlockSpec((UNITS, 3 * UNITS), lambda t: (0, 0)),
            pl.BlockSpec((1, 3 * UNITS), lambda t: (0, 0)),
            pl.BlockSpec((1, 3 * UNITS), lambda t: (0, 0)),
        ],
        out_specs=pl.BlockSpec((_TB, BATCH, UNITS), lambda t: (t, 0, 0)),
        out_shape=jax.ShapeDtypeStruct((SEQ, BATCH, UNITS), jnp.float32),
        scratch_shapes=[pltpu.VMEM((BATCH, UNITS), jnp.float32)],
        compiler_params=pltpu.CompilerParams(
            vmem_limit_bytes=100 * 1024 * 1024,
        ),
    )(emb, h0, wk, wr, bi, br)


def kernel(x, gru_init_state, embedding, kernel, recurrent_kernel, bias_input, bias_recurrent):
    idx = jnp.transpose(x.astype(jnp.int32), (1, 0)).reshape(_ROWS)
    rows = _make_sc_gather()(embedding, idx)
    emb = rows.reshape(SEQ, BATCH, EMBED_DIM)

    out_tbu = _gru_scan(
        emb,
        gru_init_state,
        kernel.astype(jnp.bfloat16),
        recurrent_kernel.astype(jnp.bfloat16),
        bias_input.reshape(1, 3 * UNITS),
        bias_recurrent.reshape(1, 3 * UNITS),
    )
    output = jnp.transpose(out_tbu, (1, 0, 2))
    state = out_tbu[SEQ - 1]
    return (output, state)


# speedup vs baseline: 2.4619x; 1.4134x over previous
import functools

import jax
import jax.numpy as jnp
from jax import lax
from jax.experimental import pallas as pl
from jax.experimental.pallas import tpu as pltpu
from jax.experimental.pallas import tpu_sc as plsc

VOCAB = 1000000
EMBED_DIM = 64
UNITS = 128
BATCH = 1024
SEQ = 50

_NC = 2
_NS = 16
_NW = _NC * _NS
_ROWS = BATCH * SEQ
_RPW = _ROWS // _NW
_FLUSH = 800
_NFL = _RPW // _FLUSH
_G = 16
_RINGG = 6


@functools.lru_cache(maxsize=1)
def _make_sc_gather():
    mesh = plsc.VectorSubcoreMesh(core_axis_name="c", subcore_axis_name="s")

    @functools.partial(
        pl.kernel,
        mesh=mesh,
        out_type=jax.ShapeDtypeStruct((_ROWS, EMBED_DIM), jnp.float32),
        scratch_types=[
            pltpu.VMEM((_RPW,), jnp.int32),
            pltpu.VMEM((_FLUSH, EMBED_DIM), jnp.float32),
            pltpu.SemaphoreType.DMA,
        ],
        compiler_params=pltpu.CompilerParams(use_tc_tiling_on_sc=True),
    )
    def sc_gather(table_hbm, idx_hbm, out_hbm, idx_v, rows_v, sem):
        wid = lax.axis_index("s") * _NC + lax.axis_index("c")
        base = wid * _RPW
        pltpu.sync_copy(idx_hbm.at[pl.ds(base, _RPW)], idx_v)

        def drain_group():
            pltpu.make_async_copy(
                table_hbm.at[0, pl.ds(0, 8)],
                rows_v.at[pl.ds(0, 8)],
                sem,
            ).wait()
            pltpu.make_async_copy(
                table_hbm.at[0, pl.ds(0, 8)],
                rows_v.at[pl.ds(0, 8)],
                sem,
            ).wait()

        for c in range(_NFL):
            cbase = c * _FLUSH

            def fire(g, carry, cbase=cbase):
                vec = idx_v[pl.ds(cbase + g * _G, _G)]
                for j in range(_G):
                    i = vec[j]
                    pltpu.async_copy(
                        table_hbm.at[i >> 3, pl.ds(i & 7, 1)],
                        rows_v.at[pl.ds(g * _G + j, 1)],
                        sem,
                    )

                @pl.when(g >= _RINGG)
                def _():
                    drain_group()

                return carry

            lax.fori_loop(0, _FLUSH // _G, fire, 0)

            for _ in range(_RINGG):
                drain_group()
            pltpu.sync_copy(rows_v, out_hbm.at[pl.ds(base + c * _FLUSH, _FLUSH)])

    return sc_gather


_TB = 8
_NTB = (SEQ + _TB - 1) // _TB


def _gru_body(emb_ref, h0_ref, wk_ref, wr_ref, bi_ref, br_ref, out_ref, h_ref):
    tb = pl.program_id(0)

    @pl.when(tb == 0)
    def _():
        h_ref[...] = h0_ref[...]

    h = h_ref[...]
    wk = wk_ref[...]
    wr = wr_ref[...]
    bi = bi_ref[...]
    br = br_ref[...]
    for t in range(_TB):
        xt = emb_ref[t]
        matx = jnp.dot(
            xt.astype(jnp.bfloat16), wk, preferred_element_type=jnp.float32
        ) + bi
        math = jnp.dot(
            h.astype(jnp.bfloat16), wr, preferred_element_type=jnp.float32
        ) + br
        xz = matx[:, 0:UNITS]
        xr = matx[:, UNITS:2 * UNITS]
        xh = matx[:, 2 * UNITS:3 * UNITS]
        hz = math[:, 0:UNITS]
        hr = math[:, UNITS:2 * UNITS]
        hh_rec = math[:, 2 * UNITS:3 * UNITS]
        z = jax.nn.sigmoid(xz + hz)
        r = jax.nn.sigmoid(xr + hr)
        hh = jnp.tanh(xh + r * hh_rec)
        h = z * h + (1.0 - z) * hh
        out_ref[t] = h
    h_ref[...] = h


def _gru_scan(emb, h0, wk, wr, bi, br):
    return pl.pallas_call(
        _gru_body,
        grid=(_NTB,),
        in_specs=[
            pl.BlockSpec((_TB, BATCH, EMBED_DIM), lambda t: (t, 0, 0)),
            pl.BlockSpec((BATCH, UNITS), lambda t: (0, 0)),
            pl.BlockSpec((EMBED_DIM, 3 * UNITS), lambda t: (0, 0)),
            pl.BlockSpec((UNITS, 3 * UNITS), lambda t: (0, 0)),
            pl.BlockSpec((1, 3 * UNITS), lambda t: (0, 0)),
            pl.BlockSpec((1, 3 * UNITS), lambda t: (0, 0)),
        ],
        out_specs=pl.BlockSpec((_TB, BATCH, UNITS), lambda t: (t, 0, 0)),
        out_shape=jax.ShapeDtypeStruct((SEQ, BATCH, UNITS), jnp.float32),
        scratch_shapes=[pltpu.VMEM((BATCH, UNITS), jnp.float32)],
        compiler_params=pltpu.CompilerParams(
            vmem_limit_bytes=100 * 1024 * 1024,
        ),
    )(emb, h0, wk, wr, bi, br)


def kernel(x, gru_init_state, embedding, kernel, recurrent_kernel, bias_input, bias_recurrent):
    idx = jnp.transpose(x.astype(jnp.int32), (1, 0)).reshape(_ROWS)
    rows = _make_sc_gather()(embedding.reshape(VOCAB // 8, 8, EMBED_DIM), idx)
    emb = rows.reshape(SEQ, BATCH, EMBED_DIM)

    out_tbu = _gru_scan(
        emb,
        gru_init_state,
        kernel.astype(jnp.bfloat16),
        recurrent_kernel.astype(jnp.bfloat16),
        bias_input.reshape(1, 3 * UNITS),
        bias_recurrent.reshape(1, 3 * UNITS),
    )
    output = jnp.transpose(out_tbu, (1, 0, 2))
    state = out_tbu[SEQ - 1]
    return (output, state)


# speedup vs baseline: 2.4896x; 1.0112x over previous
import functools

import jax
import jax.numpy as jnp
from jax import lax
from jax.experimental import pallas as pl
from jax.experimental.pallas import tpu as pltpu
from jax.experimental.pallas import tpu_sc as plsc

VOCAB = 1000000
EMBED_DIM = 64
UNITS = 128
BATCH = 1024
SEQ = 50

_NC = 2
_NS = 16
_NW = _NC * _NS
_ROWS = BATCH * SEQ
_RPW = _ROWS // _NW
_FLUSH = 800
_NFL = _RPW // _FLUSH
_G = 16
_RINGG = 8


@functools.lru_cache(maxsize=1)
def _make_sc_gather():
    mesh = plsc.VectorSubcoreMesh(core_axis_name="c", subcore_axis_name="s")

    @functools.partial(
        pl.kernel,
        mesh=mesh,
        out_type=jax.ShapeDtypeStruct((_ROWS, EMBED_DIM), jnp.float32),
        scratch_types=[
            pltpu.VMEM((_RPW,), jnp.int32),
            pltpu.VMEM((_FLUSH, EMBED_DIM), jnp.float32),
            pltpu.SemaphoreType.DMA,
        ],
        compiler_params=pltpu.CompilerParams(use_tc_tiling_on_sc=True),
    )
    def sc_gather(table_hbm, idx_hbm, out_hbm, idx_v, rows_v, sem):
        wid = lax.axis_index("s") * _NC + lax.axis_index("c")
        base = wid * _RPW
        pltpu.sync_copy(idx_hbm.at[pl.ds(base, _RPW)], idx_v)

        def drain_group():
            pltpu.make_async_copy(
                table_hbm.at[0, pl.ds(0, 8)],
                rows_v.at[pl.ds(0, 8)],
                sem,
            ).wait()
            pltpu.make_async_copy(
                table_hbm.at[0, pl.ds(0, 8)],
                rows_v.at[pl.ds(0, 8)],
                sem,
            ).wait()

        for c in range(_NFL):
            cbase = c * _FLUSH

            def fire(g, carry, cbase=cbase):
                vec = idx_v[pl.ds(cbase + g * _G, _G)]
                for j in range(_G):
                    i = vec[j]
                    pltpu.async_copy(
                        table_hbm.at[i >> 3, pl.ds(i & 7, 1)],
                        rows_v.at[pl.ds(g * _G + j, 1)],
                        sem,
                    )

                @pl.when(g >= _RINGG)
                def _():
                    drain_group()

                return carry

            lax.fori_loop(0, _FLUSH // _G, fire, 0)

            for _ in range(_RINGG):
                drain_group()
            pltpu.sync_copy(rows_v, out_hbm.at[pl.ds(base + c * _FLUSH, _FLUSH)])

    return sc_gather


_TB = 10
_NTB = SEQ // _TB


def _gru_body(emb_ref, h0_ref, wk_ref, wr_ref, bi_ref, br_ref, out_ref, h_ref):
    tb = pl.program_id(0)

    @pl.when(tb == 0)
    def _():
        h_ref[...] = h0_ref[...]

    h = h_ref[...]
    wk = wk_ref[...]
    wr = wr_ref[...]
    bi = bi_ref[...]
    br = br_ref[...]
    for t in range(_TB):
        xt = emb_ref[t]
        matx = jnp.dot(
            xt.astype(jnp.bfloat16), wk, preferred_element_type=jnp.float32
        ) + bi
        math = jnp.dot(
            h.astype(jnp.bfloat16), wr, preferred_element_type=jnp.float32
        ) + br
        xz = matx[:, 0:UNITS]
        xr = matx[:, UNITS:2 * UNITS]
        xh = matx[:, 2 * UNITS:3 * UNITS]
        hz = math[:, 0:UNITS]
        hr = math[:, UNITS:2 * UNITS]
        hh_rec = math[:, 2 * UNITS:3 * UNITS]
        z = jax.nn.sigmoid(xz + hz)
        r = jax.nn.sigmoid(xr + hr)
        hh = jnp.tanh(xh + r * hh_rec)
        h = z * h + (1.0 - z) * hh
        out_ref[t] = h
    h_ref[...] = h


def _gru_scan(emb, h0, wk, wr, bi, br):
    return pl.pallas_call(
        _gru_body,
        grid=(_NTB,),
        in_specs=[
            pl.BlockSpec((_TB, BATCH, EMBED_DIM), lambda t: (t, 0, 0)),
            pl.BlockSpec((BATCH, UNITS), lambda t: (0, 0)),
            pl.BlockSpec((EMBED_DIM, 3 * UNITS), lambda t: (0, 0)),
            pl.BlockSpec((UNITS, 3 * UNITS), lambda t: (0, 0)),
            pl.BlockSpec((1, 3 * UNITS), lambda t: (0, 0)),
            pl.BlockSpec((1, 3 * UNITS), lambda t: (0, 0)),
        ],
        out_specs=pl.BlockSpec((_TB, BATCH, UNITS), lambda t: (t, 0, 0)),
        out_shape=jax.ShapeDtypeStruct((SEQ, BATCH, UNITS), jnp.float32),
        scratch_shapes=[pltpu.VMEM((BATCH, UNITS), jnp.float32)],
        compiler_params=pltpu.CompilerParams(
            vmem_limit_bytes=100 * 1024 * 1024,
        ),
    )(emb, h0, wk, wr, bi, br)


def kernel(x, gru_init_state, embedding, kernel, recurrent_kernel, bias_input, bias_recurrent):
    idx = jnp.transpose(x.astype(jnp.int32), (1, 0)).reshape(_ROWS)
    rows = _make_sc_gather()(embedding.reshape(VOCAB // 8, 8, EMBED_DIM), idx)
    emb = rows.reshape(SEQ, BATCH, EMBED_DIM)

    out_tbu = _gru_scan(
        emb,
        gru_init_state,
        kernel.astype(jnp.bfloat16),
        recurrent_kernel.astype(jnp.bfloat16),
        bias_input.reshape(1, 3 * UNITS),
        bias_recurrent.reshape(1, 3 * UNITS),
    )
    output = jnp.transpose(out_tbu, (1, 0, 2))
    state = out_tbu[SEQ - 1]
    return (output, state)
